# Initial kernel scaffold; baseline (speedup 1.0000x reference)
import jax
import jax.numpy as jnp
from jax.experimental import pallas as pl


def kernel(word_x, word_edge_index, word_batch, word_edge_weight, sent_x, sent_edge_index, sent_batch, sent_edge_weight, W1w, b1w, W2w, b2w, Wcw, bcw, W1s, b1s, W2s, b2s, Wcs, bcs, Wf, bf):
    raise NotImplementedError("write your pallas kernel here")


# speedup vs baseline: 2.5186x; 2.5186x over previous
import functools

import jax
import jax.numpy as jnp
from jax import lax
from jax.experimental import pallas as pl
from jax.experimental.pallas import tpu as pltpu
from jax.experimental.pallas import tpu_sc as plsc

NC = 2
NS = 16
NW = NC * NS
L = 16
D = 128
NG = 64
_HI = jax.lax.Precision.HIGHEST


@functools.lru_cache(maxsize=None)
def _sc_scatter(tn: int, n_pad: int, e: int, k: int):
    assert e % NW == 0
    e_per_w = e // NW
    assert e_per_w % k == 0 and k % 16 == 0 and k <= 128
    n_chunks = e_per_w // k
    rows_per_tile = n_pad // NS
    zrows = 128
    assert rows_per_tile % zrows == 0

    mesh = plsc.VectorSubcoreMesh(
        core_axis_name="c", subcore_axis_name="s",
        num_cores=NC, num_subcores=NS)

    @functools.partial(
        pl.kernel, mesh=mesh,
        out_type=jax.ShapeDtypeStruct((NC, n_pad, D), jnp.float32),
        scratch_types=[
            pltpu.VMEM((k,), jnp.int32),
            pltpu.VMEM((k,), jnp.int32),
            pltpu.VMEM((k,), jnp.float32),
            pltpu.VMEM((k, D), jnp.float32),
            pltpu.VMEM((k, D), jnp.float32),
            pltpu.VMEM((zrows, D), jnp.float32),
            pltpu.VMEM_SHARED((n_pad, D), jnp.float32),
            pltpu.SemaphoreType.DMA,
        ],
    )
    def kern(y_hbm, src_hbm, dst_hbm, ew_hbm, out_hbm,
             src_v, dst_v, ew_v, rows_v, scaled_v, zb_v, z_sh, sem):
        cid = lax.axis_index("c")
        sid = lax.axis_index("s")
        wid = sid * NC + cid

        zvec = jnp.zeros((L,), jnp.float32)

        def zrow(r, _):
            for j in range(D // L):
                zb_v[r, pl.ds(j * L, L)] = zvec
            return 0
        lax.fori_loop(0, zrows, zrow, 0)

        def zcopy(t, _):
            pltpu.sync_copy(
                zb_v, z_sh.at[pl.ds(sid * rows_per_tile + t * zrows, zrows)])
            return 0
        lax.fori_loop(0, rows_per_tile // zrows, zcopy, 0)
        plsc.subcore_barrier()

        base = wid * e_per_w

        def chunk(i, _):
            off = base + i * k
            pltpu.sync_copy(src_hbm.at[pl.ds(off, k)], src_v)
            pltpu.sync_copy(dst_hbm.at[pl.ds(off, k)], dst_v)
            pltpu.sync_copy(ew_hbm.at[pl.ds(off, k)], ew_v)
            pltpu.async_copy(y_hbm.at[src_v], rows_v, sem).wait()

            def group(g, _):
                w16 = ew_v[pl.ds(g * L, L)]
                for j in range(L):
                    e0 = g * L + j
                    wj = w16[j]
                    for f in range(D // L):
                        sl = pl.ds(f * L, L)
                        scaled_v[e0, sl] = rows_v[e0, sl] * wj
                return 0
            lax.fori_loop(0, k // L, group, 0)

            pltpu.sync_copy(scaled_v, z_sh.at[dst_v], add=True)
            return 0
        lax.fori_loop(0, n_chunks, chunk, 0)
        plsc.subcore_barrier()

        pltpu.sync_copy(
            z_sh.at[pl.ds(sid * rows_per_tile, rows_per_tile)],
            out_hbm.at[cid, pl.ds(sid * rows_per_tile, rows_per_tile)])

    return kern


def _tc_layer_body(z_ref, w_ref, b_ref, h_ref):
    agg = z_ref[0] + z_ref[1]
    h_ref[...] = jax.nn.relu(
        jnp.dot(agg, w_ref[...], preferred_element_type=jnp.float32)
        + b_ref[...])


def _pool_head(z_ref, w2_ref, b2_ref, batch_ref, wc_ref, bc_ref):
    n = batch_ref.shape[1]
    agg = z_ref[0, :n] + z_ref[1, :n]
    h = jax.nn.relu(
        jnp.dot(agg, w2_ref[...], preferred_element_type=jnp.float32)
        + b2_ref[...])
    gids = lax.broadcasted_iota(jnp.int32, (NG, n), 0)
    mask = (gids == batch_ref[...]).astype(jnp.float32)
    sums = jnp.dot(mask, h, preferred_element_type=jnp.float32,
                   precision=_HI)
    cnt = jnp.sum(mask, axis=1, keepdims=True)
    pooled = sums / jnp.maximum(cnt, 1.0)
    return jnp.dot(pooled, wc_ref[...],
                   preferred_element_type=jnp.float32) + bc_ref[...]


def _tc_out_body(zw_ref, w2w_ref, b2w_ref, batchw_ref, wcw_ref, bcw_ref,
                 zs_ref, w2s_ref, b2s_ref, batchs_ref, wcs_ref, bcs_ref,
                 wf_ref, bf_ref, out_ref):
    xw = _pool_head(zw_ref, w2w_ref, b2w_ref, batchw_ref, wcw_ref, bcw_ref)
    xs = _pool_head(zs_ref, w2s_ref, b2s_ref, batchs_ref, wcs_ref, bcs_ref)
    out_ref[...] = jnp.dot(xw + xs, wf_ref[...],
                           preferred_element_type=jnp.float32) + bf_ref[...]


def kernel(word_x, word_edge_index, word_batch, word_edge_weight,
           sent_x, sent_edge_index, sent_batch, sent_edge_weight,
           W1w, b1w, W2w, b2w, Wcw, bcw,
           W1s, b1s, W2s, b2s, Wcs, bcs, Wf, bf):
    n_w, e_w = word_x.shape[0], word_edge_weight.shape[0]
    n_s, e_s = sent_x.shape[0], sent_edge_weight.shape[0]

    src_w, dst_w = word_edge_index[0], word_edge_index[1]
    src_s, dst_s = sent_edge_index[0], sent_edge_index[1]

    def _pad_rows(n):
        q = NS * 128
        return ((n + q - 1) // q) * q

    K_E = 80

    def _pad_edges(src, dst, ew):
        e = src.shape[0]
        q = NW * K_E
        e_pad = ((e + q - 1) // q) * q
        if e_pad == e:
            return src, dst, ew, e
        pad = e_pad - e
        zi = jnp.zeros((pad,), jnp.int32)
        return (jnp.concatenate([src, zi]), jnp.concatenate([dst, zi]),
                jnp.concatenate([ew, jnp.zeros((pad,), jnp.float32)]), e_pad)

    np_w, np_s = _pad_rows(n_w), _pad_rows(n_s)
    src_w, dst_w, ew_w, e_w = _pad_edges(src_w, dst_w, word_edge_weight)
    src_s, dst_s, ew_s, e_s = _pad_edges(src_s, dst_s, sent_edge_weight)

    z1w = _sc_scatter(n_w, np_w, e_w, K_E)(word_x, src_w, dst_w, ew_w)
    z1s = _sc_scatter(n_s, np_s, e_s, K_E)(sent_x, src_s, dst_s, ew_s)

    lay_w = pl.pallas_call(
        _tc_layer_body, out_shape=jax.ShapeDtypeStruct((np_w, D), jnp.float32))
    lay_s = pl.pallas_call(
        _tc_layer_body, out_shape=jax.ShapeDtypeStruct((np_s, D), jnp.float32))
    h1w = lay_w(z1w, W1w, b1w.reshape(1, D))
    h1s = lay_s(z1s, W1s, b1s.reshape(1, D))

    z2w = _sc_scatter(np_w, np_w, e_w, K_E)(h1w, src_w, dst_w, ew_w)
    z2s = _sc_scatter(np_s, np_s, e_s, K_E)(h1s, src_s, dst_s, ew_s)

    n_cls = Wf.shape[0]
    out = pl.pallas_call(
        _tc_out_body,
        out_shape=jax.ShapeDtypeStruct((NG, n_cls), jnp.float32),
    )(z2w, W2w, b2w.reshape(1, D), word_batch.reshape(1, n_w), Wcw,
      bcw.reshape(1, n_cls),
      z2s, W2s, b2s.reshape(1, D), sent_batch.reshape(1, n_s), Wcs,
      bcs.reshape(1, n_cls), Wf, bf.reshape(1, n_cls))
    return out
